# baseline (speedup 1.0000x reference)
import functools
import math

import jax
import jax.numpy as jnp
import numpy as np
from jax import lax
from jax.experimental import pallas as pl
from jax.experimental.pallas import tpu as pltpu
from jax.experimental.pallas import tpu_sc as plsc

VOCAB = 1000000
D_MODEL = 64
MAX_LEN = 512
BATCH = 4096
SEQ = 200
SCALE = math.sqrt(D_MODEL)

NC, NS = 2, 16
NW = NC * NS
ROWS_PER_W = BATCH // NW
NCHUNK = ROWS_PER_W
TOK_PER_W = ROWS_PER_W * SEQ
LANES = 16
NBUF = 3
WIDE = 2 * D_MODEL
IDXPAD = SEQ + LANES


def _make_pe():
    pe = np.zeros((MAX_LEN, D_MODEL), dtype=np.float32)
    position = np.arange(0, MAX_LEN, dtype=np.float32)[:, None]
    div_term = np.exp(
        np.arange(0, D_MODEL, 2, dtype=np.float32) * (-math.log(10000.0) / D_MODEL)
    )
    pe[:, 0::2] = np.sin(position * div_term)
    pe[:, 1::2] = np.cos(position * div_term)
    return pe[:SEQ]


_PE_NP = _make_pe()

_MESH = plsc.VectorSubcoreMesh(
    core_axis_name="c", subcore_axis_name="s", num_cores=NC, num_subcores=NS
)


TCOLS = 256
NFULL = VOCAB // TCOLS
TTAIL = VOCAB - NFULL * TCOLS
TPAIRS = TCOLS // 2


@functools.partial(
    pl.kernel,
    out_type=jax.ShapeDtypeStruct((VOCAB // 2, WIDE), jnp.float32),
    mesh=_MESH,
    compiler_params=pltpu.CompilerParams(needs_layout_passes=False),
    scratch_types=[
        pltpu.VMEM((D_MODEL, TCOLS), jnp.float32),
        pltpu.VMEM((D_MODEL, TCOLS), jnp.float32),
        pltpu.VMEM((128, WIDE), jnp.float32),
        pltpu.VMEM((128, WIDE), jnp.float32),
        pltpu.VMEM((D_MODEL, TTAIL), jnp.float32),
        pltpu.SemaphoreType.DMA,
        pltpu.SemaphoreType.DMA,
        pltpu.SemaphoreType.DMA,
        pltpu.SemaphoreType.DMA,
    ],
)
def _format_kernel(tT_hbm, tailT_hbm, t2_hbm,
                   src0, src1, dst0, dst1, tailv, i0, i1, o0, o1):
    wid = lax.axis_index("s") * NC + lax.axis_index("c")
    cnt = jnp.where(wid < NFULL % NW, NFULL // NW + 1, NFULL // NW)

    src = [src0, src1]
    dst = [dst0, dst1]
    isem = [i0, i1]
    osem = [o0, o1]

    d_idx = [lax.iota(jnp.int32, LANES) + q * LANES
             for q in range(D_MODEL // LANES)]

    def in_desc(it, k, width=TCOLS):
        cg = wid + it * NW
        return pltpu.make_async_copy(
            tT_hbm.at[:, pl.ds(cg * TCOLS, width)],
            src[k].at[:, pl.ds(0, width)], isem[k],
        )

    def out_desc(it, k, pairs=TPAIRS):
        cg = wid + it * NW
        return pltpu.make_async_copy(
            dst[k].at[pl.ds(0, pairs)],
            t2_hbm.at[pl.ds(cg * TPAIRS, pairs)],
            osem[k],
        )

    def transpose(s, d, pairs):
        @plsc.parallel_loop(0, pairs, unroll=2)
        def _tr(p):
            v0 = jnp.broadcast_to(2 * p, (LANES,)).astype(jnp.int32)
            v1 = v0 + 1
            for q in range(D_MODEL // LANES):
                d[p, pl.ds(q * LANES, LANES)] = plsc.load_gather(
                    s, [d_idx[q], v0]
                )
                d[p, pl.ds(D_MODEL + q * LANES, LANES)] = plsc.load_gather(
                    s, [d_idx[q], v1]
                )

    def slot(it, k):
        in_desc(it, k).wait()

        @pl.when(it >= 2)
        def _():
            out_desc(it - 2, k).wait()

        transpose(src[k], dst[k], TPAIRS)
        out_desc(it, k).start()

        @pl.when(it + 2 < cnt)
        def _():
            in_desc(it + 2, k).start()

    in_desc(0, 0).start()
    in_desc(1, 1).start()

    nmin = NFULL // NW

    @pl.loop(0, nmin // 2)
    def _pipe2(it):
        slot(2 * it, 0)
        slot(2 * it + 1, 1)

    @pl.when(cnt > nmin)
    def _():
        slot(nmin, nmin % 2)
        out_desc(nmin - 1, (nmin - 1) % 2).wait()
        out_desc(nmin, nmin % 2).wait()

    @pl.when(cnt == nmin)
    def _():
        out_desc(nmin - 2, nmin % 2).wait()
        out_desc(nmin - 1, (nmin - 1) % 2).wait()

    @pl.when(wid == NW - 1)
    def _tail():
        pltpu.sync_copy(tailT_hbm, tailv)
        transpose(tailv, dst0, TTAIL // 2)
        pltpu.sync_copy(
            dst0.at[pl.ds(0, TTAIL // 2)],
            t2_hbm.at[pl.ds(NFULL * TPAIRS, TTAIL // 2)],
        )


@functools.partial(
    pl.kernel,
    out_type=jax.ShapeDtypeStruct((BATCH, SEQ, WIDE), jnp.float32),
    mesh=_MESH,
    compiler_params=pltpu.CompilerParams(use_tc_tiling_on_sc=False),
    scratch_types=[
        pltpu.VMEM((IDXPAD,), jnp.int32),
        pltpu.VMEM((IDXPAD,), jnp.int32),
        pltpu.VMEM((IDXPAD,), jnp.int32),
        pltpu.VMEM((SEQ,), jnp.int32),
        pltpu.VMEM((SEQ,), jnp.int32),
        pltpu.VMEM((SEQ,), jnp.int32),
        pltpu.VMEM((SEQ, WIDE), jnp.float32),
        pltpu.VMEM((SEQ, WIDE), jnp.float32),
        pltpu.VMEM((SEQ, WIDE), jnp.float32),
        pltpu.VMEM((SEQ, D_MODEL), jnp.float32),
        pltpu.SemaphoreType.DMA,
        pltpu.SemaphoreType.DMA,
        pltpu.SemaphoreType.DMA,
        pltpu.SemaphoreType.DMA,
        pltpu.SemaphoreType.DMA,
        pltpu.SemaphoreType.DMA,
        pltpu.SemaphoreType.DMA,
        pltpu.SemaphoreType.DMA,
        pltpu.SemaphoreType.DMA,
    ],
)
def _embed_kernel(table2_hbm, xf_hbm, xfh_hbm, pe_hbm, out_hbm,
                  iv0, iv1, iv2, ih0, ih1, ih2, rows0, rows1, rows2, pe_v,
                  s0, s1, s2, g0, g1, g2, w0, w1, w2):
    wid = lax.axis_index("s") * NC + lax.axis_index("c")
    row0 = wid * ROWS_PER_W
    tok0 = row0 * SEQ

    idxv = [iv0, iv1, iv2]
    idxh = [ih0, ih1, ih2]
    rows = [rows0, rows1, rows2]
    isem = [s0, s1, s2]
    gsem = [g0, g1, g2]
    wsem = [w0, w1, w2]

    pltpu.sync_copy(pe_hbm, pe_v)

    def idxv_desc(c, k):
        return pltpu.make_async_copy(
            xf_hbm.at[pl.ds(tok0 + c * SEQ, SEQ)],
            idxv[k].at[pl.ds(0, SEQ)], isem[k],
        )

    def idxh_desc(c, k):
        return pltpu.make_async_copy(
            xfh_hbm.at[pl.ds(tok0 + c * SEQ, SEQ)], idxh[k], isem[k]
        )

    def fire_idx(c, k):
        idxv_desc(c, k).start()
        idxh_desc(c, k).start()

    def gather_desc(c, k):
        src = table2_hbm.at[idxh[k]]
        return pltpu.make_async_copy(src, rows[k], gsem[k])

    def fire_gather(c, k):
        idxv_desc(c, k).wait()
        idxh_desc(c, k).wait()
        gather_desc(c, k).start()

    def wb_desc(c, k):
        return pltpu.make_async_copy(rows[k], out_hbm.at[row0 + c], wsem[k])

    def process(c, k):
        gather_desc(c, k).wait()

        @plsc.parallel_loop(0, SEQ, unroll=2)
        def _ew(j):
            buf = rows[k]
            vvec = idxv[k][pl.ds(j, LANES)]
            odd = (vvec[0] & 1) == 1

            @pl.when(odd)
            def _():
                for q in range(D_MODEL // LANES):
                    sl = pl.ds(q * LANES, LANES)
                    hi = pl.ds(D_MODEL + q * LANES, LANES)
                    buf[j, sl] = buf[j, hi] * SCALE + pe_v[j, sl]

            @pl.when(jnp.logical_not(odd))
            def _():
                for q in range(D_MODEL // LANES):
                    sl = pl.ds(q * LANES, LANES)
                    buf[j, sl] = buf[j, sl] * SCALE + pe_v[j, sl]

        wb_desc(c, k).start()

    def drain_wb(c, k):
        wb_desc(c, k).wait()

    fire_idx(0, 0)
    fire_idx(1, 1)
    fire_idx(2, 2)
    fire_gather(0, 0)
    fire_gather(1, 1)

    process(0, 0)
    fire_idx(3, 0)
    fire_gather(2, 2)

    process(1, 1)
    fire_idx(4, 1)
    drain_wb(0, 0)
    fire_gather(3, 0)

    process(2, 2)
    fire_idx(5, 2)
    drain_wb(1, 1)
    fire_gather(4, 1)

    @pl.loop(1, (NCHUNK - 2) // NBUF)
    def _pipe(it):
        cbase = it * NBUF
        for k in range(NBUF):
            c = cbase + k
            process(c, k)

            @pl.when(c + 3 < NCHUNK)
            def _():
                fire_idx(c + 3, k)

            drain_wb(c - 1, (k + NBUF - 1) % NBUF)

            @pl.when(c + 2 < NCHUNK)
            def _():
                fire_gather(c + 2, (k + 2) % NBUF)

    process(NCHUNK - 2, (NCHUNK - 2) % NBUF)
    drain_wb(NCHUNK - 3, (NCHUNK - 3) % NBUF)
    process(NCHUNK - 1, (NCHUNK - 1) % NBUF)
    drain_wb(NCHUNK - 2, (NCHUNK - 2) % NBUF)
    drain_wb(NCHUNK - 1, (NCHUNK - 1) % NBUF)


def kernel(x, table):
    tT = table.T
    table2 = _format_kernel(tT, tT[:, NFULL * TCOLS:])
    xf = x.reshape(BATCH * SEQ)
    xfh = lax.shift_right_logical(xf, 1)
    out = _embed_kernel(table2, xf, xfh, jnp.asarray(_PE_NP))
    return out[:, :, :D_MODEL]


# speedup vs baseline: 1.1746x; 1.1746x over previous
import functools
import math

import jax
import jax.numpy as jnp
import numpy as np
from jax import lax
from jax.experimental import pallas as pl
from jax.experimental.pallas import tpu as pltpu
from jax.experimental.pallas import tpu_sc as plsc

VOCAB = 1000000
D_MODEL = 64
MAX_LEN = 512
BATCH = 4096
SEQ = 200
SCALE = math.sqrt(D_MODEL)

NC, NS = 2, 16
NW = NC * NS
ROWS_PER_W = BATCH // NW
NCHUNK = ROWS_PER_W
TOK_PER_W = ROWS_PER_W * SEQ
LANES = 16
NBUF = 3
WIDE = 2 * D_MODEL
IDXPAD = SEQ + LANES


def _make_pe():
    pe = np.zeros((MAX_LEN, D_MODEL), dtype=np.float32)
    position = np.arange(0, MAX_LEN, dtype=np.float32)[:, None]
    div_term = np.exp(
        np.arange(0, D_MODEL, 2, dtype=np.float32) * (-math.log(10000.0) / D_MODEL)
    )
    pe[:, 0::2] = np.sin(position * div_term)
    pe[:, 1::2] = np.cos(position * div_term)
    return pe[:SEQ]


_PE_NP = _make_pe()

_MESH = plsc.VectorSubcoreMesh(
    core_axis_name="c", subcore_axis_name="s", num_cores=NC, num_subcores=NS
)


@functools.partial(
    pl.kernel,
    out_type=jax.ShapeDtypeStruct((BATCH, SEQ, WIDE), jnp.float32),
    mesh=_MESH,
    compiler_params=pltpu.CompilerParams(
        use_tc_tiling_on_sc=False, disable_bounds_checks=True
    ),
    scratch_types=[
        pltpu.VMEM((IDXPAD,), jnp.int32),
        pltpu.VMEM((IDXPAD,), jnp.int32),
        pltpu.VMEM((IDXPAD,), jnp.int32),
        pltpu.VMEM((SEQ,), jnp.int32),
        pltpu.VMEM((SEQ,), jnp.int32),
        pltpu.VMEM((SEQ,), jnp.int32),
        pltpu.VMEM((SEQ, WIDE), jnp.float32),
        pltpu.VMEM((SEQ, WIDE), jnp.float32),
        pltpu.VMEM((SEQ, WIDE), jnp.float32),
        pltpu.VMEM((SEQ, D_MODEL), jnp.float32),
        pltpu.SemaphoreType.DMA,
        pltpu.SemaphoreType.DMA,
        pltpu.SemaphoreType.DMA,
        pltpu.SemaphoreType.DMA,
        pltpu.SemaphoreType.DMA,
        pltpu.SemaphoreType.DMA,
        pltpu.SemaphoreType.DMA,
        pltpu.SemaphoreType.DMA,
        pltpu.SemaphoreType.DMA,
    ],
)
def _embed_kernel(table2_hbm, xf_hbm, xfh_hbm, pe_hbm, out_hbm,
                  iv0, iv1, iv2, ih0, ih1, ih2, rows0, rows1, rows2, pe_v,
                  s0, s1, s2, g0, g1, g2, w0, w1, w2):
    wid = lax.axis_index("s") * NC + lax.axis_index("c")
    row0 = wid * ROWS_PER_W
    tok0 = row0 * SEQ

    idxv = [iv0, iv1, iv2]
    idxh = [ih0, ih1, ih2]
    rows = [rows0, rows1, rows2]
    isem = [s0, s1, s2]
    gsem = [g0, g1, g2]
    wsem = [w0, w1, w2]

    pltpu.sync_copy(pe_hbm, pe_v)

    def idxv_desc(c, k):
        return pltpu.make_async_copy(
            xf_hbm.at[pl.ds(tok0 + c * SEQ, SEQ)],
            idxv[k].at[pl.ds(0, SEQ)], isem[k],
        )

    def idxh_desc(c, k):
        return pltpu.make_async_copy(
            xfh_hbm.at[pl.ds(tok0 + c * SEQ, SEQ)], idxh[k], isem[k]
        )

    def fire_idx(c, k):
        idxv_desc(c, k).start()
        idxh_desc(c, k).start()

    def gather_desc(c, k):
        src = table2_hbm.at[idxh[k]]
        return pltpu.make_async_copy(src, rows[k], gsem[k])

    def fire_gather(c, k):
        idxv_desc(c, k).wait()
        idxh_desc(c, k).wait()
        gather_desc(c, k).start()

    def wb_desc(c, k):
        return pltpu.make_async_copy(rows[k], out_hbm.at[row0 + c], wsem[k])

    def process(c, k):
        gather_desc(c, k).wait()

        @plsc.parallel_loop(0, SEQ, unroll=2)
        def _ew(j):
            buf = rows[k]
            vvec = idxv[k][pl.ds(j, LANES)]
            odd = (vvec[0] & 1) == 1

            @pl.when(odd)
            def _():
                for q in range(D_MODEL // LANES):
                    sl = pl.ds(q * LANES, LANES)
                    hi = pl.ds(D_MODEL + q * LANES, LANES)
                    buf[j, sl] = buf[j, hi] * SCALE + pe_v[j, sl]

            @pl.when(jnp.logical_not(odd))
            def _():
                for q in range(D_MODEL // LANES):
                    sl = pl.ds(q * LANES, LANES)
                    buf[j, sl] = buf[j, sl] * SCALE + pe_v[j, sl]

        wb_desc(c, k).start()

    def drain_wb(c, k):
        wb_desc(c, k).wait()

    fire_idx(0, 0)
    fire_idx(1, 1)
    fire_idx(2, 2)
    fire_gather(0, 0)
    fire_gather(1, 1)

    process(0, 0)
    fire_idx(3, 0)
    fire_gather(2, 2)

    process(1, 1)
    fire_idx(4, 1)
    drain_wb(0, 0)
    fire_gather(3, 0)

    process(2, 2)
    fire_idx(5, 2)
    drain_wb(1, 1)
    fire_gather(4, 1)

    @pl.loop(1, (NCHUNK - 2) // NBUF)
    def _pipe(it):
        cbase = it * NBUF
        for k in range(NBUF):
            c = cbase + k
            process(c, k)

            @pl.when(c + 3 < NCHUNK)
            def _():
                fire_idx(c + 3, k)

            drain_wb(c - 1, (k + NBUF - 1) % NBUF)

            @pl.when(c + 2 < NCHUNK)
            def _():
                fire_gather(c + 2, (k + 2) % NBUF)

    process(NCHUNK - 2, (NCHUNK - 2) % NBUF)
    drain_wb(NCHUNK - 3, (NCHUNK - 3) % NBUF)
    process(NCHUNK - 1, (NCHUNK - 1) % NBUF)
    drain_wb(NCHUNK - 2, (NCHUNK - 2) % NBUF)
    drain_wb(NCHUNK - 1, (NCHUNK - 1) % NBUF)


def kernel(x, table):
    table2 = table.reshape(VOCAB // 2, WIDE)
    xf = x.reshape(BATCH * SEQ)
    xfh = lax.shift_right_logical(xf, 1)
    out = _embed_kernel(table2, xf, xfh, jnp.asarray(_PE_NP))
    return out[:, :, :D_MODEL]
